# Initial kernel scaffold; baseline (speedup 1.0000x reference)
import jax
import jax.numpy as jnp
from jax.experimental import pallas as pl


def kernel(h, edge_index, W1, b1, g1, be1, W2, b2, g2, be2, g3, be3):
    raise NotImplementedError("write your pallas kernel here")


# speedup vs baseline: 2.0538x; 2.0538x over previous
import functools

import jax
import jax.numpy as jnp
from jax import lax
from jax.experimental import pallas as pl
from jax.experimental.pallas import tpu as pltpu
from jax.experimental.pallas import tpu_sc as plsc

N = 10000
E = 320000
D = 128
L = 5

NC = 2
NS = 16
NW = NC * NS

def _half_bounds():
    b = [0]
    for _ in range(11):
        b.append(b[-1] + 10080)
    for _ in range(4):
        b.append(b[-1] + 9840)
    b.append(b[-1] + 9760)
    return b

_HB = _half_bounds()
BND = _HB[:-1] + [160000 + x for x in _HB]

LOCROWS = 768
NOUT = LOCROWS // 128
NDUMP = 48
CSTREAM = 128


def _iota16():
    return lax.iota(jnp.int32, 16)


def _seg_sum_body(x_hbm, src_hbm, dst_hbm, meta_hbm, main_hbm, bnd_hbm,
                  srcbuf, dstbuf, rows_v, loc, meta_v, outidx, idx1, tmp16,
                  sem):
    cid = lax.axis_index("c")
    sid = lax.axis_index("s")
    wid = cid * NS + sid

    j = wid % 16
    half = wid // 16
    base = half * (E // 2) + jnp.where(
        j <= 11, j * 10080, 110880 + (j - 11) * 9840)
    clen = jnp.where(j < 11, 10080, jnp.where(j < 15, 9840, 9760))
    n128 = clen // CSTREAM

    pltpu.sync_copy(meta_hbm.at[pl.ds(wid * 16, 16)], meta_v)
    locbase_v = meta_v[...]
    pltpu.sync_copy(meta_hbm.at[pl.ds(NW * 16 + wid * 16, 16)], tmp16)
    rlo_v = tmp16[...]

    z16 = jnp.zeros((16,), jnp.float32)

    def zrow(r, c):
        loc[pl.ds(r * 16, 16)] = z16
        return c

    lax.fori_loop(0, LOCROWS * 8, zrow, 0)

    def accum_groups(ngroups):
        iota = _iota16()
        for g in range(ngroups):
            dv = dstbuf[pl.ds(g * 16, 16)]
            rloc = jnp.minimum(dv - locbase_v, LOCROWS - 1)
            for e in range(16):
                rsplat = jax.lax.gather(
                    rloc,
                    jnp.full((16, 1), e, jnp.int32),
                    jax.lax.GatherDimensionNumbers(
                        offset_dims=(), collapsed_slice_dims=(0,),
                        start_index_map=(0,)),
                    (1,),
                    mode=jax.lax.GatherScatterMode.PROMISE_IN_BOUNDS)
                rbase = rsplat * D
                for q in range(8):
                    vals = rows_v[g * 16 + e, pl.ds(q * 16, 16)]
                    plsc.addupdate_scatter(
                        loc, [rbase + (iota + q * 16)], vals)

    def chunk_body(ci, c):
        b = base + ci * CSTREAM
        pltpu.sync_copy(src_hbm.at[pl.ds(b, CSTREAM)], srcbuf)
        pltpu.sync_copy(dst_hbm.at[pl.ds(b, CSTREAM)], dstbuf)
        pltpu.async_copy(x_hbm.at[srcbuf], rows_v, sem).wait()
        accum_groups(CSTREAM // 16)
        return c

    lax.fori_loop(0, n128, chunk_body, 0)

    def tail(tl):
        b = base + n128 * CSTREAM
        pltpu.sync_copy(src_hbm.at[pl.ds(b, tl)], srcbuf.at[pl.ds(0, tl)])
        pltpu.sync_copy(dst_hbm.at[pl.ds(b, tl)], dstbuf.at[pl.ds(0, tl)])
        pltpu.async_copy(x_hbm.at[srcbuf.at[pl.ds(0, tl)]],
                         rows_v.at[pl.ds(0, tl)], sem).wait()
        accum_groups(tl // 16)

    @pl.when(j < 11)
    def _():
        tail(96)

    @pl.when(jnp.logical_and(j >= 11, j < 15))
    def _():
        tail(112)

    @pl.when(j == 15)
    def _():
        tail(32)

    pltpu.sync_copy(meta_hbm.at[pl.ds(2 * NW * 16 + wid * 16, 16)], tmp16)
    rhi_v = tmp16[...]
    for jj in range(NOUT):
        for q in range(8):
            pos = _iota16() + (jj * 128 + q * 16)
            valid = jnp.logical_and(pos >= rlo_v, pos < rhi_v)
            idx = jnp.where(valid, locbase_v + pos, N + (pos & 31))
            outidx[jj, pl.ds(q * 16, 16)] = idx
    for jj in range(NOUT):
        def stage(r, c):
            for q in range(8):
                rows_v[r, pl.ds(q * 16, 16)] = loc[
                    pl.ds((jj * 128 + r) * D + q * 16, 16)]
            return c

        lax.fori_loop(0, 128, stage, 0)
        pltpu.async_copy(rows_v, main_hbm.at[outidx.at[jj]], sem).wait()

    for q in range(8):
        rows_v[0, pl.ds(q * 16, 16)] = loc[pl.ds(q * 16, 16)]
    plsc.store_scatter(idx1, [jnp.zeros((16,), jnp.int32)],
                       jnp.full((16,), wid, jnp.int32),
                       mask=_iota16() == 0)
    pltpu.async_copy(rows_v.at[pl.ds(0, 1)], bnd_hbm.at[idx1], sem).wait()


_seg_sum = functools.partial(
    pl.kernel,
    out_type=(jax.ShapeDtypeStruct((N + NDUMP, D), jnp.float32),
              jax.ShapeDtypeStruct((NW, D), jnp.float32)),
    mesh=plsc.VectorSubcoreMesh(core_axis_name="c", subcore_axis_name="s"),
    compiler_params=pltpu.CompilerParams(needs_layout_passes=False),
    scratch_types=[
        pltpu.VMEM((CSTREAM,), jnp.int32),
        pltpu.VMEM((CSTREAM,), jnp.int32),
        pltpu.VMEM((CSTREAM, D), jnp.float32),
        pltpu.VMEM((LOCROWS * D,), jnp.float32),
        pltpu.VMEM((16,), jnp.int32),
        pltpu.VMEM((NOUT, 128), jnp.int32),
        pltpu.VMEM((1,), jnp.int32),
        pltpu.VMEM((16,), jnp.int32),
        pltpu.SemaphoreType.DMA,
    ],
)(_seg_sum_body)


def _tree_mean(acc):
    a = acc[0:4] + acc[4:8]
    a = a[0:2] + a[2:4]
    s = a[0:1] + a[1:2]
    return s * jnp.float32(1.0 / N)


def _bn_relu(t, tref, g, be):
    tref[...] = t.reshape(N // 8, 8, D)

    def body(i, acc):
        return acc + tref[i]

    m = _tree_mean(lax.fori_loop(0, N // 8, body,
                                 jnp.zeros((8, D), jnp.float32)))
    d = t - m
    tref[...] = (d * d).reshape(N // 8, 8, D)
    v = _tree_mean(lax.fori_loop(0, N // 8, body,
                                 jnp.zeros((8, D), jnp.float32)))
    return jnp.maximum(d / jnp.sqrt(v + jnp.float32(1e-5)) * g + be,
                       jnp.float32(0.0))


def _dense_body(x_ref, main_ref, bnd_ref, oh_ref, W1_ref, b1_ref, g1_ref,
                be1_ref, W2_ref, b2_ref, g2_ref, be2_ref, g3_ref, be3_ref,
                out_ref, tref):
    agg = main_ref[...] + jax.lax.dot(
        oh_ref[...], bnd_ref[...], precision=lax.Precision.HIGHEST,
        preferred_element_type=jnp.float32)
    rst = x_ref[...] + agg
    t = jnp.dot(rst.astype(jnp.bfloat16), W1_ref[...].astype(jnp.bfloat16),
                preferred_element_type=jnp.float32) + b1_ref[...]
    t = _bn_relu(t, tref, g1_ref[...], be1_ref[...])
    t = jnp.dot(t.astype(jnp.bfloat16), W2_ref[...].astype(jnp.bfloat16),
                preferred_element_type=jnp.float32) + b2_ref[...]
    t = _bn_relu(t, tref, g2_ref[...], be2_ref[...])
    out_ref[...] = _bn_relu(t, tref, g3_ref[...], be3_ref[...])


_dense = pl.pallas_call(
    _dense_body,
    out_shape=jax.ShapeDtypeStruct((N, D), jnp.float32),
    scratch_shapes=[pltpu.VMEM((N // 8, 8, D), jnp.float32)],
)


def kernel(h, edge_index, W1, b1, g1, be1, W2, b2, g2, be2, g3, be3):
    src = edge_index[0]
    dst = edge_index[1]

    perm = jnp.argsort(dst, stable=True)
    src_s = src[perm]
    dst_s = dst[perm]
    bnd_arr = jnp.array(BND[:-1], jnp.int32)
    ids = dst_s[bnd_arr]
    last = dst_s[jnp.array([b - 1 for b in BND[1:]], jnp.int32)]
    straddle = jnp.concatenate(
        [jnp.zeros((1,), jnp.int32),
         (ids[1:] == last[:-1]).astype(jnp.int32)])
    w = ids + straddle
    w = w.at[0].set(0)
    locbase = ids.at[0].set(0)
    wnext = jnp.concatenate([w[1:], jnp.array([N], jnp.int32)])
    rlo = w - locbase
    rhi = wnext - locbase
    meta = jnp.broadcast_to(
        jnp.stack([locbase, rlo, rhi])[:, :, None], (3, NW, 16)
    ).reshape(-1).astype(jnp.int32)
    onehot = (
        (jnp.arange(N, dtype=jnp.int32)[:, None] == ids[None, :])
        & (straddle[None, :] == 1)
    ).astype(jnp.float32)

    x = h
    for i in range(L):
        main_ext, bnd = _seg_sum(x, src_s, dst_s, meta)
        x = _dense(x, main_ext[:N], bnd, onehot, W1[i], b1[i], g1[i], be1[i],
                   W2[i], b2[i], g2[i], be2[i], g3[i], be3[i])
    return x
